# baseline (speedup 1.0000x reference)
import functools

import jax
import jax.numpy as jnp
from jax.experimental import pallas as pl
from jax.experimental.pallas import tpu as pltpu

_N_E = 1024
_E_DIM = 64
_BETA = 0.25
_BLOCK = 2048


def _vq_body(z_ref, cb_ref, zq_ref, idx_ref, acc_ref):
    i = pl.program_id(0)
    z = z_ref[...]
    cb = cb_ref[...]
    z2 = jnp.sum(z * z, axis=1, keepdims=True)
    e2 = jnp.sum(cb * cb, axis=1, keepdims=True).T
    mm = jax.lax.dot_general(z, cb, (((1,), (1,)), ((), ())),
                             preferred_element_type=jnp.float32)
    dist = z2 + e2 - 2.0 * mm
    mind = jnp.min(dist, axis=1, keepdims=True)
    col = jax.lax.broadcasted_iota(jnp.int32, dist.shape, 1)
    idx = jnp.min(jnp.where(dist == mind, col, _N_E), axis=1)
    onehot = (col == idx[:, None]).astype(jnp.float32)
    g = jax.lax.dot_general(onehot, cb, (((1,), (0,)), ((), ())),
                            preferred_element_type=jnp.float32,
                            precision=jax.lax.Precision.HIGHEST)
    zq_ref[...] = z + (g - z)
    idx_ref[...] = idx.reshape(idx_ref.shape)

    @pl.when(i == 0)
    def _init():
        acc_ref[...] = jnp.zeros_like(acc_ref)

    acc_ref[...] += jnp.sum((g - z) ** 2).reshape(1, 1)


@jax.jit
def kernel(z, codebook):
    n, d = z.shape
    n_e = codebook.shape[0]
    nb = n // _BLOCK
    zq, idx3, acc = pl.pallas_call(
        _vq_body,
        grid=(nb,),
        in_specs=[
            pl.BlockSpec((_BLOCK, d), lambda i: (i, 0)),
            pl.BlockSpec((n_e, d), lambda i: (0, 0)),
        ],
        out_specs=[
            pl.BlockSpec((_BLOCK, d), lambda i: (i, 0)),
            pl.BlockSpec((1, 1, _BLOCK), lambda i: (i, 0, 0)),
            pl.BlockSpec((1, 1), lambda i: (0, 0)),
        ],
        out_shape=[
            jax.ShapeDtypeStruct((n, d), jnp.float32),
            jax.ShapeDtypeStruct((nb, 1, _BLOCK), jnp.int32),
            jax.ShapeDtypeStruct((1, 1), jnp.float32),
        ],
    )(z, codebook)
    indices = idx3.reshape(n)
    mean_sq = acc[0, 0] / (n * d)
    loss = mean_sq + _BETA * mean_sq
    return (zq, loss, indices)


# speedup vs baseline: 1.6045x; 1.6045x over previous
import functools

import jax
import jax.numpy as jnp
from jax import lax
from jax.experimental import pallas as pl
from jax.experimental.pallas import tpu as pltpu
from jax.experimental.pallas import tpu_sc as plsc

_N_E = 1024
_E_DIM = 64
_BETA = 0.25
_BLOCK = 2048

_NC = 2
_NS = 16
_NW = _NC * _NS
_CHUNK = 128


def _vq_body(z_ref, cb_ref, idx_ref, acc_ref):
    i = pl.program_id(0)
    z = z_ref[...]
    cb = cb_ref[...]
    z2 = jnp.sum(z * z, axis=1, keepdims=True)
    e2 = jnp.sum(cb * cb, axis=1, keepdims=True).T
    mm = lax.dot_general(z, cb, (((1,), (1,)), ((), ())),
                         preferred_element_type=jnp.float32)
    dist = z2 + e2 - 2.0 * mm
    mind = jnp.min(dist, axis=1, keepdims=True)
    col = lax.broadcasted_iota(jnp.int32, dist.shape, 1)
    idx = jnp.min(jnp.where(dist == mind, col, _N_E), axis=1)
    idx_ref[...] = idx.reshape(idx_ref.shape)

    @pl.when(i == 0)
    def _init():
        acc_ref[...] = jnp.zeros_like(acc_ref)

    acc_ref[...] += jnp.sum(mind).reshape(1, 1)


def _gather_body(cbp_hbm, idx_hbm, out_hbm, idx_v, rows_v, sem):
    wid = lax.axis_index("s") * _NC + lax.axis_index("c")
    nch = idx_v.shape[0]
    bpw = nch * _CHUNK
    half = bpw // 2
    pltpu.sync_copy(idx_hbm.at[pl.ds(wid * nch, nch)], idx_v)
    for p in range(2):
        copies = [pltpu.async_copy(cbp_hbm.at[idx_v.at[p * (nch // 2) + j]],
                                   rows_v.at[pl.ds(j * _CHUNK, _CHUNK)], sem)
                  for j in range(nch // 2)]
        for c in copies:
            c.wait()
        pltpu.sync_copy(rows_v, out_hbm.at[pl.ds(wid * bpw + p * half, half)])


@jax.jit
def kernel(z, codebook):
    n, d = z.shape
    n_e = codebook.shape[0]
    nb = n // _BLOCK
    idx3, acc = pl.pallas_call(
        _vq_body,
        grid=(nb,),
        in_specs=[
            pl.BlockSpec((_BLOCK, d), lambda i: (i, 0)),
            pl.BlockSpec((n_e, d), lambda i: (0, 0)),
        ],
        out_specs=[
            pl.BlockSpec((1, 1, _BLOCK), lambda i: (i, 0, 0)),
            pl.BlockSpec((1, 1), lambda i: (0, 0)),
        ],
        out_shape=[
            jax.ShapeDtypeStruct((nb, 1, _BLOCK), jnp.int32),
            jax.ShapeDtypeStruct((1, 1), jnp.float32),
        ],
    )(z, codebook)
    indices = idx3.reshape(n)

    bpw = n // _NW
    nch = bpw // _CHUNK
    cbp = jnp.concatenate(
        [codebook, jnp.zeros((n_e, 128 - d), jnp.float32)], axis=1)
    mesh = plsc.VectorSubcoreMesh(core_axis_name="c", subcore_axis_name="s")
    gather = pl.kernel(
        _gather_body,
        mesh=mesh,
        out_type=jax.ShapeDtypeStruct((n, 128), jnp.float32),
        scratch_types=[
            pltpu.VMEM((nch, _CHUNK), jnp.int32),
            pltpu.VMEM((bpw // 2, 128), jnp.float32),
            pltpu.SemaphoreType.DMA,
        ],
    )
    zq = gather(cbp, indices.reshape(n // _CHUNK, _CHUNK))[:, :d]

    mean_sq = acc[0, 0] / (n * d)
    loss = mean_sq + _BETA * mean_sq
    return (zq, loss, indices)
